# baseline (speedup 1.0000x reference)
import jax
import jax.numpy as jnp
from jax.experimental import pallas as pl
from jax.experimental.pallas import tpu as pltpu

_CH = 2048
_NBUF = 6
_NLOOK = 5


def _mm_kernel(x_ref, w_hbm, o_ref, wbufs, wsems):
    i = pl.program_id(0)
    nfull = w_hbm.shape[0] // _CH
    tail = w_hbm.shape[0] - nfull * _CH

    def full_copy(chunk, slot):
        return pltpu.make_async_copy(
            w_hbm.at[pl.ds(chunk * _CH, _CH), :], wbufs.at[slot], wsems.at[slot])

    def tail_copy(slot):
        return pltpu.make_async_copy(
            w_hbm.at[pl.ds(nfull * _CH, tail), :],
            wbufs.at[slot, pl.ds(0, tail), :], wsems.at[slot])

    def start_fetch(chunk, slot):
        @pl.when(chunk < nfull)
        def _():
            full_copy(chunk, slot).start()

        @pl.when(chunk == nfull)
        def _():
            tail_copy(slot).start()

    @pl.when(i == 0)
    def _():
        for j in range(_NLOOK):
            start_fetch(jnp.int32(j), jnp.int32(j))

    slot = jax.lax.rem(i, _NBUF)

    @pl.when(i < nfull)
    def _():
        full_copy(i, slot).wait()

    @pl.when(i == nfull)
    def _():
        tail_copy(slot).wait()

    o_ref[:, 0, :] = jax.lax.dot_general(
        x_ref[:, 0, :], wbufs[slot],
        dimension_numbers=(((1,), (1,)), ((), ())),
        preferred_element_type=jnp.float32,
    )

    nxt = i + _NLOOK
    start_fetch(nxt, jax.lax.rem(nxt, _NBUF))


def kernel(hidden_states, W):
    B, Q, H = hidden_states.shape
    V = W.shape[0]
    return pl.pallas_call(
        _mm_kernel,
        grid=(pl.cdiv(V, _CH),),
        in_specs=[
            pl.BlockSpec((B, Q, H), lambda i: (0, 0, 0)),
            pl.BlockSpec(memory_space=pl.ANY),
        ],
        out_specs=pl.BlockSpec((B, Q, _CH), lambda i: (0, 0, i)),
        out_shape=jax.ShapeDtypeStruct((B, Q, V), jnp.float32),
        scratch_shapes=[
            pltpu.VMEM((_NBUF, _CH, H), jnp.float32),
            pltpu.SemaphoreType.DMA((_NBUF,)),
        ],
    )(hidden_states, W)


# speedup vs baseline: 1.0186x; 1.0186x over previous
import jax
import jax.numpy as jnp
from jax.experimental import pallas as pl

_BV = 5120


def _mm_kernel(x_ref, w_ref, o_ref):
    o_ref[:, 0, :] = jax.lax.dot_general(
        x_ref[:, 0, :], w_ref[...],
        dimension_numbers=(((1,), (1,)), ((), ())),
        preferred_element_type=jnp.float32,
    )


def kernel(hidden_states, W):
    B, Q, H = hidden_states.shape
    V = W.shape[0]
    last_valid = (V - 1) // _BV
    return pl.pallas_call(
        _mm_kernel,
        grid=(pl.cdiv(V, _BV),),
        in_specs=[
            pl.BlockSpec((B, Q, H), lambda i: (0, 0, 0)),
            pl.BlockSpec((_BV, H), lambda i: (jnp.minimum(i, last_valid), 0)),
        ],
        out_specs=pl.BlockSpec((B, Q, _BV), lambda i: (0, 0, i)),
        out_shape=jax.ShapeDtypeStruct((B, Q, V), jnp.float32),
    )(hidden_states, W)


# speedup vs baseline: 1.0224x; 1.0038x over previous
import jax
import jax.numpy as jnp
from jax.experimental import pallas as pl

_BV = 4096


def _mm_kernel(x_ref, w_ref, o_ref):
    o_ref[:, 0, :] = jax.lax.dot_general(
        x_ref[:, 0, :], w_ref[...],
        dimension_numbers=(((1,), (1,)), ((), ())),
        preferred_element_type=jnp.float32,
    )


def kernel(hidden_states, W):
    B, Q, H = hidden_states.shape
    V = W.shape[0]
    last_valid = (V - 1) // _BV
    return pl.pallas_call(
        _mm_kernel,
        grid=(pl.cdiv(V, _BV),),
        in_specs=[
            pl.BlockSpec((B, Q, H), lambda i: (0, 0, 0)),
            pl.BlockSpec((_BV, H), lambda i: (jnp.minimum(i, last_valid), 0)),
        ],
        out_specs=pl.BlockSpec((B, Q, _BV), lambda i: (0, 0, i)),
        out_shape=jax.ShapeDtypeStruct((B, Q, V), jnp.float32),
    )(hidden_states, W)
